# baseline (speedup 1.0000x reference)
import functools

import jax
import jax.numpy as jnp
from jax import lax
from jax.experimental import pallas as pl
from jax.experimental.pallas import tpu as pltpu
from jax.experimental.pallas import tpu_sc as plsc

_NC = 2
_NS = 16
_NW = _NC * _NS
_CHUNK = 128
_NBUF = 6
_PF = 4


@functools.lru_cache(maxsize=None)
def _build(n_chunks: int, d: int):
    b_total = _NW * n_chunks * _CHUNK
    rows_per_w = n_chunks * _CHUNK
    mesh = plsc.VectorSubcoreMesh(
        core_axis_name="c", subcore_axis_name="s",
        num_cores=_NC, num_subcores=_NS,
    )

    @functools.partial(
        pl.kernel,
        out_type=jax.ShapeDtypeStruct((b_total, d), jnp.float32),
        mesh=mesh,
        compiler_params=pltpu.CompilerParams(use_tc_tiling_on_sc=True),
        scratch_types=[
            pltpu.VMEM((rows_per_w,), jnp.int32),
            pltpu.VMEM((_NBUF, _CHUNK, d), jnp.float32),
            [pltpu.SemaphoreType.DMA] * _NBUF,
            [pltpu.SemaphoreType.DMA] * _NBUF,
        ],
    )
    def embed(table_hbm, idx_hbm, out_hbm, idx_v, rows_v, gsems, wsems):
        wid = lax.axis_index("s") * _NC + lax.axis_index("c")
        base = wid * rows_per_w
        pltpu.sync_copy(idx_hbm.at[pl.ds(base, rows_per_w)], idx_v)

        def issue_g(jj, b):
            pltpu.async_copy(
                table_hbm.at[idx_v.at[pl.ds(jj * _CHUNK, _CHUNK)]],
                rows_v.at[b], gsems[b],
            )

        def wait_g(b):
            pltpu.make_async_copy(
                table_hbm.at[idx_v.at[pl.ds(0, _CHUNK)]], rows_v.at[b],
                gsems[b],
            ).wait()

        def issue_w(jj, b):
            pltpu.async_copy(
                rows_v.at[b], out_hbm.at[pl.ds(base + jj * _CHUNK, _CHUNK)],
                wsems[b],
            )

        def wait_w(b):
            pltpu.make_async_copy(
                rows_v.at[b], out_hbm.at[pl.ds(base, _CHUNK)], wsems[b]
            ).wait()

        for c in range(_PF):
            issue_g(c, c % _NBUF)
        for jj in range(n_chunks):
            if jj + _PF - _NBUF >= 0:
                wait_w((jj + _PF) % _NBUF)
            if jj + _PF < n_chunks:
                issue_g(jj + _PF, (jj + _PF) % _NBUF)
            wait_g(jj % _NBUF)
            issue_w(jj, jj % _NBUF)
        for jj in range(max(0, n_chunks - (_NBUF - _PF)), n_chunks):
            wait_w(jj % _NBUF)

    return embed


def kernel(x, table):
    bt, h = x.shape
    v, d = table.shape
    b_total = bt * h
    n_chunks = b_total // (_NW * _CHUNK)
    idx = x.T.reshape(-1).astype(jnp.int32)
    out = _build(n_chunks, d)(table, idx)
    return out.reshape(h, bt, d).transpose(1, 0, 2)


# speedup vs baseline: 1.0422x; 1.0422x over previous
import functools

import jax
import jax.numpy as jnp
from jax import lax
from jax.experimental import pallas as pl
from jax.experimental.pallas import tpu as pltpu
from jax.experimental.pallas import tpu_sc as plsc

_NC = 2
_NS = 16
_NW = _NC * _NS
_CHUNK = 128
_NBUF = 7
_PF = 6


@functools.lru_cache(maxsize=None)
def _build(n_chunks: int, d: int):
    _E = _NBUF - _PF
    assert n_chunks >= _NBUF + _E and (n_chunks - _NBUF - _E) % _NBUF == 0
    b_total = _NW * n_chunks * _CHUNK
    rows_per_w = n_chunks * _CHUNK
    mesh = plsc.VectorSubcoreMesh(
        core_axis_name="c", subcore_axis_name="s",
        num_cores=_NC, num_subcores=_NS,
    )

    @functools.partial(
        pl.kernel,
        out_type=jax.ShapeDtypeStruct((b_total, d), jnp.float32),
        mesh=mesh,
        compiler_params=pltpu.CompilerParams(use_tc_tiling_on_sc=True),
        scratch_types=[
            pltpu.VMEM((rows_per_w,), jnp.int32),
            pltpu.VMEM((_NBUF, _CHUNK, d), jnp.float32),
            [pltpu.SemaphoreType.DMA] * _NBUF,
            [pltpu.SemaphoreType.DMA] * _NBUF,
        ],
    )
    def embed(table_hbm, idx_hbm, out_hbm, idx_v, rows_v, gsems, wsems):
        wid = lax.axis_index("s") * _NC + lax.axis_index("c")
        base = wid * rows_per_w
        pltpu.sync_copy(idx_hbm.at[pl.ds(base, rows_per_w)], idx_v)

        def issue_g(jj, b):
            pltpu.async_copy(
                table_hbm.at[idx_v.at[pl.ds(jj * _CHUNK, _CHUNK)]],
                rows_v.at[b], gsems[b],
            )

        def wait_g(b):
            pltpu.make_async_copy(
                table_hbm.at[idx_v.at[pl.ds(0, _CHUNK)]], rows_v.at[b],
                gsems[b],
            ).wait()

        def issue_w(jj, b):
            pltpu.async_copy(
                rows_v.at[b], out_hbm.at[pl.ds(base + jj * _CHUNK, _CHUNK)],
                wsems[b],
            )

        def wait_w(b):
            pltpu.make_async_copy(
                rows_v.at[b], out_hbm.at[pl.ds(base, _CHUNK)], wsems[b]
            ).wait()

        for c in range(_PF):
            issue_g(c, c)
        for c in range(_E):
            issue_g(c + _PF, (c + _PF) % _NBUF)
            wait_g(c)
            issue_w(c, c)

        @pl.loop(_E, n_chunks - _NBUF, step=_NBUF)
        def _(j):
            for b in range(_NBUF):
                jj = j + b
                b_refill = (b + _E + _PF) % _NBUF
                b_cur = (b + _E) % _NBUF
                wait_w(b_refill)
                issue_g(jj + _PF, b_refill)
                wait_g(b_cur)
                issue_w(jj, b_cur)

        for jj in range(n_chunks - _NBUF, n_chunks):
            wait_w((jj + _PF) % _NBUF)
            if jj + _PF < n_chunks:
                issue_g(jj + _PF, (jj + _PF) % _NBUF)
            wait_g(jj % _NBUF)
            issue_w(jj, jj % _NBUF)
        for jj in range(n_chunks - _E, n_chunks):
            wait_w(jj % _NBUF)

    return embed


def kernel(x, table):
    bt, h = x.shape
    v, d = table.shape
    b_total = bt * h
    n_chunks = b_total // (_NW * _CHUNK)
    idx = x.T.reshape(-1).astype(jnp.int32)
    out = _build(n_chunks, d)(table, idx)
    return out.reshape(h, bt, d).transpose(1, 0, 2)
